# baseline (speedup 1.0000x reference)
import jax
import jax.numpy as jnp
from jax import lax
from jax.experimental import pallas as pl
from jax.experimental.pallas import tpu as pltpu

_H, _W, _C = 495, 436, 8
_HW = _W * _C
_OFFSETS = (12, 13, 14, 17, 20, 23)


def _body(idx_ref, ts_ref, stats_ref, out_ref):
    i = pl.program_id(0)
    flat = idx_ref[i]
    w = flat // 32
    yt = lax.rem(flat, 32)
    r7 = lax.broadcasted_iota(jnp.int32, (7, 32), 0)
    c32 = lax.broadcasted_iota(jnp.int32, (7, 32), 1)
    sel = (r7 == w) & (c32 == yt)
    v = jnp.sum(jnp.where(sel, ts_ref[0], 0.0))
    s = jnp.sum(jnp.where(sel, ts_ref[1], 0.0))
    lane = lax.broadcasted_iota(jnp.int32, (1, _H, _HW), 2)
    scale = jnp.where(lane % 2 == 0, v, s)
    prod = stats_ref[...] * scale
    out_ref[...] = prod.astype(jnp.int32).astype(jnp.float32)


def kernel(x, additional_data, stats, timeshift_arr):
    del x
    b = additional_data.shape[0]
    weekday = additional_data[:, 0]
    time = additional_data[:, 1]
    offs = jnp.asarray(_OFFSETS, dtype=additional_data.dtype)
    y_times = time[:, None] + offs[None, :]
    flat_rows = (weekday[:, None] * 32 + y_times).reshape(-1)
    n = b * 6

    stats3 = stats.reshape(7 * 32, _H, _HW)

    grid_spec = pltpu.PrefetchScalarGridSpec(
        num_scalar_prefetch=1,
        grid=(n,),
        in_specs=[
            pl.BlockSpec((2, 7, 32), lambda i, idx: (0, 0, 0)),
            pl.BlockSpec((1, _H, _HW), lambda i, idx: (idx[i], 0, 0)),
        ],
        out_specs=pl.BlockSpec((1, _H, _HW), lambda i, idx: (i, 0, 0)),
    )
    out = pl.pallas_call(
        _body,
        grid_spec=grid_spec,
        out_shape=jax.ShapeDtypeStruct((n, _H, _HW), jnp.float32),
    )(flat_rows.astype(jnp.int32), timeshift_arr, stats3)
    return out.reshape(b, 6, _H, _W, _C)
